# Initial kernel scaffold; baseline (speedup 1.0000x reference)
import jax
import jax.numpy as jnp
from jax.experimental import pallas as pl


def kernel(hidden_states, router_w, w_gate, w_up, w_down, shared_w_gate, shared_w_up, shared_w_down):
    raise NotImplementedError("write your pallas kernel here")


# speedup vs baseline: 1.5295x; 1.5295x over previous
import functools

import jax
import jax.numpy as jnp
from jax.experimental import pallas as pl

T = 2048
D = 1024
E = 8
FF = 1024
SFF = 2048

ROUTER_TM = 512
EXPERT_TM = 512
SHARED_TM = 512


def _router_kernel(x_ref, rw_ref, xs_ref, idx_ref):
    x = x_ref[...]
    logits = jnp.dot(x, rw_ref[...].T, preferred_element_type=jnp.float32,
                     precision=jax.lax.Precision.HIGHEST)
    top = jnp.max(logits, axis=1, keepdims=True)
    ids = jax.lax.broadcasted_iota(jnp.int32, logits.shape, 1)
    idx = jnp.min(jnp.where(logits == top, ids, E), axis=1, keepdims=True)
    score = jax.nn.sigmoid(top)
    xs_ref[...] = x * score
    idx_ref[...] = idx


def _expert_kernel(xs_ref, idx_ref, wg_ref, wu_ref, wd_ref, out_ref):
    e = pl.program_id(0)

    @pl.when(e == 0)
    def _init():
        out_ref[...] = jnp.zeros_like(out_ref)

    wg = wg_ref[...].astype(jnp.bfloat16)
    wu = wu_ref[...].astype(jnp.bfloat16)
    wd = wd_ref[...].astype(jnp.bfloat16)

    def body(i, _):
        base = i * EXPERT_TM
        xb = xs_ref[pl.ds(base, EXPERT_TM), :].astype(jnp.bfloat16)
        g = jnp.dot(xb, wg, preferred_element_type=jnp.float32)
        u = jnp.dot(xb, wu, preferred_element_type=jnp.float32)
        h = (jax.nn.silu(g) * u).astype(jnp.bfloat16)
        y = jnp.dot(h, wd, preferred_element_type=jnp.float32)
        mask = (idx_ref[pl.ds(base, EXPERT_TM), :] == e).astype(jnp.float32)
        out_ref[pl.ds(base, EXPERT_TM), :] += y * mask
        return 0

    jax.lax.fori_loop(0, T // EXPERT_TM, body, 0)


def _shared_kernel(x_ref, routed_ref, wg_ref, wu_ref, wd_ref, out_ref):
    xb = x_ref[...].astype(jnp.bfloat16)
    wg = wg_ref[...].astype(jnp.bfloat16)
    wu = wu_ref[...].astype(jnp.bfloat16)
    wd = wd_ref[...].astype(jnp.bfloat16)
    g = jnp.dot(xb, wg, preferred_element_type=jnp.float32)
    u = jnp.dot(xb, wu, preferred_element_type=jnp.float32)
    h = (jax.nn.silu(g) * u).astype(jnp.bfloat16)
    y = jnp.dot(h, wd, preferred_element_type=jnp.float32)
    out_ref[...] = y + routed_ref[...]


@functools.partial(jax.jit, static_argnames=())
def kernel(hidden_states, router_w, w_gate, w_up, w_down,
           shared_w_gate, shared_w_up, shared_w_down):
    x_scaled, idx = pl.pallas_call(
        _router_kernel,
        grid=(T // ROUTER_TM,),
        in_specs=[
            pl.BlockSpec((ROUTER_TM, D), lambda t: (t, 0)),
            pl.BlockSpec((E, D), lambda t: (0, 0)),
        ],
        out_specs=[
            pl.BlockSpec((ROUTER_TM, D), lambda t: (t, 0)),
            pl.BlockSpec((ROUTER_TM, 1), lambda t: (t, 0)),
        ],
        out_shape=[
            jax.ShapeDtypeStruct((T, D), jnp.float32),
            jax.ShapeDtypeStruct((T, 1), jnp.int32),
        ],
    )(hidden_states, router_w)

    routed = pl.pallas_call(
        _expert_kernel,
        grid=(E,),
        in_specs=[
            pl.BlockSpec((T, D), lambda e: (0, 0)),
            pl.BlockSpec((T, 1), lambda e: (0, 0)),
            pl.BlockSpec((None, D, FF), lambda e: (e, 0, 0)),
            pl.BlockSpec((None, D, FF), lambda e: (e, 0, 0)),
            pl.BlockSpec((None, FF, D), lambda e: (e, 0, 0)),
        ],
        out_specs=pl.BlockSpec((T, D), lambda e: (0, 0)),
        out_shape=jax.ShapeDtypeStruct((T, D), jnp.float32),
    )(x_scaled, idx, w_gate, w_up, w_down)

    out = pl.pallas_call(
        _shared_kernel,
        grid=(T // SHARED_TM,),
        in_specs=[
            pl.BlockSpec((SHARED_TM, D), lambda t: (t, 0)),
            pl.BlockSpec((SHARED_TM, D), lambda t: (t, 0)),
            pl.BlockSpec((D, SFF), lambda t: (0, 0)),
            pl.BlockSpec((D, SFF), lambda t: (0, 0)),
            pl.BlockSpec((SFF, D), lambda t: (0, 0)),
        ],
        out_specs=pl.BlockSpec((SHARED_TM, D), lambda t: (t, 0)),
        out_shape=jax.ShapeDtypeStruct((T, D), jnp.float32),
    )(hidden_states, routed, shared_w_gate, shared_w_up, shared_w_down)

    return out
